# baseline (speedup 1.0000x reference)
import jax
import jax.numpy as jnp
from jax import lax
from jax.experimental import pallas as pl
from jax.experimental.pallas import tpu as pltpu
from jax.experimental.pallas import tpu_sc as plsc

F = 9
V = 150
D = 128
W = D // 2
NC = 2
NS = 16
NW = NC * NS
C = 112
K = 28
RPW = C * K
N = 100000
TW = F * V * W
IW = C * F
IWB = IW + 16
B = 4
U = 4


def _bcast_lane(vec, f):
    idx = jnp.full((16, 1), f, jnp.int32)
    dn = lax.GatherDimensionNumbers(offset_dims=(), collapsed_slice_dims=(0,),
                                    start_index_map=(0,))
    return lax.gather(vec, idx, dn, (1,),
                      mode=lax.GatherScatterMode.PROMISE_IN_BOUNDS)


def _body(idx_hbm, tab_hbm, out_hbm, tab_v, idx0, idx1, ob0, ob1,
          sem_i0, sem_i1, sem_o0, sem_o1):
    wid = lax.axis_index("s") * NC + lax.axis_index("c")
    pltpu.sync_copy(tab_hbm, tab_v)

    idx_bufs = (idx0, idx1)
    ob_bufs = (ob0, ob1)
    sem_i = (sem_i0, sem_i1)
    sem_o = (sem_o0, sem_o1)

    def rbase(k):
        return jnp.minimum(wid * RPW + k * C, N - C)

    pltpu.sync_copy(idx_hbm.at[pl.ds(rbase(0) * F, IW)],
                    idx0.at[pl.ds(0, IW)])

    lane = lax.iota(jnp.int32, 16)
    offv = lane * (V * W)
    lane16 = [lane + 16 * blk for blk in range(B)]

    def compute_chunk(idx_v, obuf):
        def rowgrp(i, rcarry):
            iva = []
            for u in range(U):
                r = i * U + u
                iva.append(idx_v[pl.ds(r * F, 16)] * W + offv)
            for blk in range(B):
                for u in range(U):
                    r = i * U + u
                    g = []
                    for f in range(F):
                        bc = _bcast_lane(iva[u], f)
                        g.append(plsc.bitcast(
                            plsc.load_gather(tab_v, [bc + lane16[blk]]),
                            jnp.bfloat16))
                    t01 = g[0] + g[1]
                    t23 = g[2] + g[3]
                    t45 = g[4] + g[5]
                    t67 = g[6] + g[7]
                    s = (t01 + t23) + (t45 + t67) + g[8]
                    lo, hi = plsc.unpack(s,
                                         format=plsc.PackFormat.INTERLEAVED)
                    obuf[r, pl.ds(blk * 16, 16)] = lo
                    obuf[r, pl.ds(W + blk * 16, 16)] = hi
            return rcarry

        lax.fori_loop(0, C // U, rowgrp, 0)

    def pair(kk, carry):
        for b in range(2):
            k = kk * 2 + b
            rb = rbase(k)

            @pl.when(k + 1 < K)
            def _prefetch():
                pltpu.async_copy(idx_hbm.at[pl.ds(rbase(k + 1) * F, IW)],
                                 idx_bufs[1 - b].at[pl.ds(0, IW)],
                                 sem_i[1 - b])

            @pl.when(k > 0)
            def _wait_idx():
                pltpu.make_async_copy(idx_hbm.at[pl.ds(rb * F, IW)],
                                      idx_bufs[b].at[pl.ds(0, IW)],
                                      sem_i[b]).wait()

            @pl.when(kk > 0)
            def _wait_out():
                pltpu.make_async_copy(ob_bufs[b], out_hbm.at[pl.ds(0, C)],
                                      sem_o[b]).wait()

            compute_chunk(idx_bufs[b], ob_bufs[b])
            pltpu.async_copy(ob_bufs[b], out_hbm.at[pl.ds(rb, C)], sem_o[b])
        return carry

    lax.fori_loop(0, K // 2, pair, 0)
    pltpu.make_async_copy(ob0, out_hbm.at[pl.ds(0, C)], sem_o0).wait()
    pltpu.make_async_copy(ob1, out_hbm.at[pl.ds(0, C)], sem_o1).wait()


def kernel(x, tables):
    n = x.shape[0]
    x_flat = x.astype(jnp.int32).reshape(n * F)

    tb = tables.astype(jnp.bfloat16).reshape(F * V, 2, W)
    tu = jax.lax.bitcast_convert_type(tb, jnp.uint16).astype(jnp.uint32)
    tword = (tu[:, 1, :] << 16) | tu[:, 0, :]
    tpack = jax.lax.bitcast_convert_type(tword, jnp.int32).reshape(TW)

    run = pl.kernel(
        _body,
        out_type=jax.ShapeDtypeStruct((n, D), jnp.float32),
        mesh=plsc.VectorSubcoreMesh(core_axis_name="c", subcore_axis_name="s"),
        compiler_params=pltpu.CompilerParams(needs_layout_passes=False,
                                             use_tc_tiling_on_sc=False),
        scratch_types=[
            pltpu.VMEM((TW,), jnp.int32),
            pltpu.VMEM((IWB,), jnp.int32),
            pltpu.VMEM((IWB,), jnp.int32),
            pltpu.VMEM((C, D), jnp.float32),
            pltpu.VMEM((C, D), jnp.float32),
            pltpu.SemaphoreType.DMA,
            pltpu.SemaphoreType.DMA,
            pltpu.SemaphoreType.DMA,
            pltpu.SemaphoreType.DMA,
        ],
    )
    return run(x_flat, tpack)


# speedup vs baseline: 1.0153x; 1.0153x over previous
import jax
import jax.numpy as jnp
from jax import lax
from jax.experimental import pallas as pl
from jax.experimental.pallas import tpu as pltpu
from jax.experimental.pallas import tpu_sc as plsc

F = 9
V = 150
D = 128
W = D // 2
NC = 2
NS = 16
NW = NC * NS
C = 112
K = 28
RPW = C * K
N = 100000
TW = F * V * W
IW = C * F
IWB = IW + 16
B = 4
U = 4


def _bcast_lane(vec, f):
    idx = jnp.full((16, 1), f, jnp.int32)
    dn = lax.GatherDimensionNumbers(offset_dims=(), collapsed_slice_dims=(0,),
                                    start_index_map=(0,))
    return lax.gather(vec, idx, dn, (1,),
                      mode=lax.GatherScatterMode.PROMISE_IN_BOUNDS)


def _body(idx_hbm, tab_hbm, out_hbm, tab_v, idx0, idx1, ob0, ob1,
          sem_i0, sem_i1, sem_o0, sem_o1):
    wid = lax.axis_index("s") * NC + lax.axis_index("c")
    pltpu.sync_copy(tab_hbm, tab_v)

    idx_bufs = (idx0, idx1)
    ob_bufs = (ob0, ob1)
    sem_i = (sem_i0, sem_i1)
    sem_o = (sem_o0, sem_o1)

    def rbase(k):
        return jnp.minimum(wid * RPW + k * C, N - C)

    pltpu.sync_copy(idx_hbm.at[pl.ds(rbase(0) * F, IW)],
                    idx0.at[pl.ds(0, IW)])

    lane = lax.iota(jnp.int32, 16)
    offv = lane * (V * W)
    lane16 = [lane + 16 * blk for blk in range(B)]

    def compute_chunk(idx_v, obuf):
        def rowgrp(i, rcarry):
            iva = []
            for u in range(U):
                r = i * U + u
                iva.append(idx_v[pl.ds(r * F, 16)] * W + offv)
            for blk in range(B):
                for u in range(U):
                    r = i * U + u
                    g = []
                    for f in range(F):
                        bc = _bcast_lane(iva[u], f)
                        g.append(plsc.bitcast(
                            plsc.load_gather(tab_v, [bc + lane16[blk]]),
                            jnp.bfloat16))
                    t01 = g[0] + g[1]
                    t23 = g[2] + g[3]
                    t45 = g[4] + g[5]
                    t67 = g[6] + g[7]
                    s = (t01 + t23) + (t45 + t67) + g[8]
                    lo, hi = plsc.unpack(s,
                                         format=plsc.PackFormat.INTERLEAVED)
                    obuf[r, pl.ds(blk * 16, 16)] = lo
                    obuf[r, pl.ds(W + blk * 16, 16)] = hi
            return rcarry

        lax.fori_loop(0, C // U, rowgrp, 0)

    def pair(kk, carry):
        for b in range(2):
            k = kk * 2 + b
            rb = rbase(k)

            @pl.when(k + 1 < K)
            def _prefetch():
                pltpu.async_copy(idx_hbm.at[pl.ds(rbase(k + 1) * F, IW)],
                                 idx_bufs[1 - b].at[pl.ds(0, IW)],
                                 sem_i[1 - b])

            @pl.when(k > 0)
            def _wait_idx():
                pltpu.make_async_copy(idx_hbm.at[pl.ds(rb * F, IW)],
                                      idx_bufs[b].at[pl.ds(0, IW)],
                                      sem_i[b]).wait()

            @pl.when(kk > 0)
            def _wait_out():
                pltpu.make_async_copy(ob_bufs[b], out_hbm.at[pl.ds(0, C)],
                                      sem_o[b]).wait()

            compute_chunk(idx_bufs[b], ob_bufs[b])
            pltpu.async_copy(ob_bufs[b], out_hbm.at[pl.ds(rb, C)], sem_o[b])
        return carry

    lax.fori_loop(0, K // 2, pair, 0)
    pltpu.make_async_copy(ob0, out_hbm.at[pl.ds(0, C)], sem_o0).wait()
    pltpu.make_async_copy(ob1, out_hbm.at[pl.ds(0, C)], sem_o1).wait()


def kernel(x, tables):
    n = x.shape[0]
    x_flat = x.astype(jnp.int32).reshape(n * F)

    tb = tables.astype(jnp.bfloat16).reshape(F * V, 2, W)
    tpair = jnp.stack([tb[:, 0, :], tb[:, 1, :]], axis=-1)
    tpack = jax.lax.bitcast_convert_type(tpair, jnp.int32).reshape(TW)

    run = pl.kernel(
        _body,
        out_type=jax.ShapeDtypeStruct((n, D), jnp.float32),
        mesh=plsc.VectorSubcoreMesh(core_axis_name="c", subcore_axis_name="s"),
        compiler_params=pltpu.CompilerParams(needs_layout_passes=False,
                                             use_tc_tiling_on_sc=False),
        scratch_types=[
            pltpu.VMEM((TW,), jnp.int32),
            pltpu.VMEM((IWB,), jnp.int32),
            pltpu.VMEM((IWB,), jnp.int32),
            pltpu.VMEM((C, D), jnp.float32),
            pltpu.VMEM((C, D), jnp.float32),
            pltpu.SemaphoreType.DMA,
            pltpu.SemaphoreType.DMA,
            pltpu.SemaphoreType.DMA,
            pltpu.SemaphoreType.DMA,
        ],
    )
    return run(x_flat, tpack)
